# baseline (speedup 1.0000x reference)
import functools

import jax
import jax.numpy as jnp
from jax import lax
from jax.experimental import pallas as pl
from jax.experimental.pallas import tpu as pltpu
from jax.experimental.pallas import tpu_sc as plsc

N = 10000
E = 160000
D = 384
H = 128
FH = 1

NC = 2
NS = 16
NW = NC * NS
C = 128
EP = 163840
NCHUNK = EP // C
SUP = 8
NSUP = NCHUNK // NW // SUP
ZR_BIG = 632
ZR_SMALL = 624


BLK1 = 2000


def _proj_body(x_ref, w_ref, o_ref):
    o_ref[...] = jnp.dot(x_ref[...], w_ref[...],
                         preferred_element_type=jnp.float32)


def _proj(e_i, w):
    return pl.pallas_call(
        _proj_body,
        grid=(N // BLK1,),
        in_specs=[
            pl.BlockSpec((BLK1, D), lambda i: (i, 0)),
            pl.BlockSpec((D, H), lambda i: (0, 0)),
        ],
        out_specs=pl.BlockSpec((BLK1, H), lambda i: (i, 0)),
        out_shape=jax.ShapeDtypeStruct((N, H), jnp.float32),
    )(e_i, w)


def _wap_body(a_ref, b_ref, o_ref):
    o_ref[...] = jnp.dot(a_ref[...], b_ref[...],
                         preferred_element_type=jnp.float32)


def _wap(W_a2, W_p1):
    return pl.pallas_call(
        _wap_body,
        out_shape=jax.ShapeDtypeStruct((H, H), jnp.float32),
    )(W_a2, W_p1)


_GATHER_DNUMS = lax.GatherDimensionNumbers(
    offset_dims=(), collapsed_slice_dims=(0,), start_index_map=(0,))


def _sc_body(g_hbm, src_hbm, dst_hbm, w_hbm, z_hbm, out_hbm,
             srcb, dstb, wb, rowsA, rowsB, acc,
             gsem0, gsem1, ssem0, ssem1):
    cid = lax.axis_index("c")
    sid = lax.axis_index("s")
    wid = sid * NC + cid

    base_big = sid * ZR_BIG
    base_small = 2 * ZR_BIG + (sid - 2) * ZR_SMALL

    @pl.when(sid < 2)
    def _():
        pltpu.sync_copy(z_hbm, acc.at[pl.ds(pl.multiple_of(base_big, 8),
                                            ZR_BIG)])

    @pl.when(sid >= 2)
    def _():
        pltpu.sync_copy(z_hbm.at[pl.ds(0, ZR_SMALL)],
                        acc.at[pl.ds(pl.multiple_of(base_small, 8), ZR_SMALL)])

    plsc.subcore_barrier()

    rows2 = (rowsA, rowsB)
    gsem2 = (gsem0, gsem1)
    ssem2 = (ssem0, ssem1)

    def scale(buf, j):
        def grp(i, c2):
            w16 = wb[j, pl.ds(i * 16, 16)]
            for jj in range(16):
                wbc = lax.gather(
                    w16, jnp.full((16, 1), jj, jnp.int32), _GATHER_DNUMS, (1,),
                    mode=lax.GatherScatterMode.PROMISE_IN_BOUNDS)
                for kk in range(H // 16):
                    sl = buf[i * 16 + jj, pl.ds(kk * 16, 16)]
                    buf[i * 16 + jj, pl.ds(kk * 16, 16)] = sl * wbc
            return c2

        lax.fori_loop(0, C // 16, grp, 0)

    def super_body(s, carry):
        csup = pl.multiple_of((wid * NSUP + s) * SUP, SUP)
        pltpu.sync_copy(src_hbm.at[pl.ds(csup, SUP)], srcb)
        pltpu.sync_copy(dst_hbm.at[pl.ds(csup, SUP)], dstb)
        pltpu.sync_copy(w_hbm.at[pl.ds(csup, SUP)], wb)
        pltpu.async_copy(g_hbm.at[srcb.at[0]], rowsA, gsem0)
        for j in range(SUP):
            cur = j % 2
            nxt = (j + 1) % 2
            pltpu.make_async_copy(g_hbm.at[srcb.at[j]], rows2[cur],
                                  gsem2[cur]).wait()
            scale(rows2[cur], j)
            if j + 1 < SUP:
                if j >= 1:
                    pltpu.make_async_copy(rows2[nxt], acc.at[dstb.at[j - 1]],
                                          ssem2[nxt]).wait()
                pltpu.async_copy(g_hbm.at[srcb.at[j + 1]], rows2[nxt],
                                 gsem2[nxt])
            pltpu.async_copy(rows2[cur], acc.at[dstb.at[j]], ssem2[cur],
                             add=True)
        pltpu.make_async_copy(rows2[0], acc.at[dstb.at[SUP - 2]],
                              ssem2[0]).wait()
        pltpu.make_async_copy(rows2[1], acc.at[dstb.at[SUP - 1]],
                              ssem2[1]).wait()
        return carry

    lax.fori_loop(0, NSUP, super_body, 0)
    plsc.subcore_barrier()

    @pl.when(sid < 2)
    def _():
        b = pl.multiple_of(base_big, 8)
        pltpu.sync_copy(acc.at[pl.ds(b, ZR_BIG)],
                        out_hbm.at[cid, pl.ds(b, ZR_BIG)])

    @pl.when(sid >= 2)
    def _():
        b = pl.multiple_of(base_small, 8)
        pltpu.sync_copy(acc.at[pl.ds(b, ZR_SMALL)],
                        out_hbm.at[cid, pl.ds(b, ZR_SMALL)])


@functools.cache
def _get_sc_segsum():
    mesh = plsc.VectorSubcoreMesh(core_axis_name="c", subcore_axis_name="s")
    return pl.kernel(
        _sc_body,
        mesh=mesh,
        out_type=jax.ShapeDtypeStruct((NC, N, H), jnp.float32),
        scratch_types=[
            pltpu.VMEM((SUP, C), jnp.int32),
            pltpu.VMEM((SUP, C), jnp.int32),
            pltpu.VMEM((SUP, C), jnp.float32),
            pltpu.VMEM((C, H), jnp.float32),
            pltpu.VMEM((C, H), jnp.float32),
            pltpu.VMEM_SHARED((N, H), jnp.float32),
            pltpu.SemaphoreType.DMA,
            pltpu.SemaphoreType.DMA,
            pltpu.SemaphoreType.DMA,
            pltpu.SemaphoreType.DMA,
        ],
    )


BLK2 = 2000


def _tc2_body(a0_ref, a1_ref, q_ref, bg1_ref, wa1_ref, ba1_ref,
              wap_ref, bp1_ref, wp2_ref, bp2_ref, out_ref):
    h = jnp.maximum(a0_ref[...] + a1_ref[...] + bg1_ref[...], 0.0)
    z = jnp.maximum(
        jnp.dot(h, wa1_ref[...], preferred_element_type=jnp.float32)
        + ba1_ref[...], 0.0)
    p = jnp.maximum(
        q_ref[...]
        + jnp.dot(z, wap_ref[...], preferred_element_type=jnp.float32)
        + bp1_ref[...], 0.0)
    out_ref[...] = (jnp.dot(p, wp2_ref[...], preferred_element_type=jnp.float32)
                    + bp2_ref[...])


def _tc2(a0, a1, q, b_g1, W_a1, b_a1, wap, b_p1, W_p2, b_p2):
    row = lambda i: (i, 0)
    full = lambda i: (0, 0)
    return pl.pallas_call(
        _tc2_body,
        grid=(N // BLK2,),
        in_specs=[
            pl.BlockSpec((BLK2, H), row),
            pl.BlockSpec((BLK2, H), row),
            pl.BlockSpec((BLK2, H), row),
            pl.BlockSpec((1, H), full),
            pl.BlockSpec((H, H), full),
            pl.BlockSpec((1, H), full),
            pl.BlockSpec((H, H), full),
            pl.BlockSpec((1, H), full),
            pl.BlockSpec((H, FH), full),
            pl.BlockSpec((1, FH), full),
        ],
        out_specs=pl.BlockSpec((BLK2, FH), row),
        out_shape=jax.ShapeDtypeStruct((N, FH), jnp.float32),
    )(a0, a1, q, b_g1, W_a1, b_a1, wap, b_p1, W_p2, b_p2)


def kernel(e_i, edge_index, edge_weight, W_g1, b_g1, W_a1, b_a1, W_a2,
           W_p1, b_p1, W_p2, b_p2):
    g = _proj(e_i, W_g1)
    q = _proj(e_i, W_p1)
    wap = _wap(W_a2, W_p1)

    zrows = jnp.zeros((ZR_BIG, H), jnp.float32)
    pad = EP - E
    spread = jnp.arange(pad, dtype=jnp.int32)
    src_p = jnp.concatenate([edge_index[0], spread]).reshape(NCHUNK, C)
    dst_p = jnp.concatenate([edge_index[1], spread]).reshape(NCHUNK, C)
    w_p = jnp.pad(edge_weight, (0, pad)).reshape(NCHUNK, C)
    parts = _get_sc_segsum()(g, src_p, dst_p, w_p, zrows)

    return _tc2(parts[0], parts[1], q,
                b_g1.reshape(1, H), W_a1, b_a1.reshape(1, H), wap,
                b_p1.reshape(1, H), W_p2, b_p2.reshape(1, FH))


# speedup vs baseline: 1.1056x; 1.1056x over previous
import functools

import jax
import jax.numpy as jnp
from jax import lax
from jax.experimental import pallas as pl
from jax.experimental.pallas import tpu as pltpu
from jax.experimental.pallas import tpu_sc as plsc

N = 10000
E = 160000
D = 384
H = 128
FH = 1

NC = 2
NS = 16
NW = NC * NS
C = 128
EP = 163840
NCHUNK = EP // C
SUP = 8
NSUP = NCHUNK // NW // SUP
ZR_BIG = 632
ZR_SMALL = 624


BLK1 = 2000


def _proj_body(x_ref, w_ref, o_ref):
    o_ref[...] = jnp.dot(x_ref[...], w_ref[...],
                         preferred_element_type=jnp.float32)


def _proj(e_i, w):
    return pl.pallas_call(
        _proj_body,
        grid=(N // BLK1,),
        in_specs=[
            pl.BlockSpec((BLK1, D), lambda i: (i, 0)),
            pl.BlockSpec((D, H), lambda i: (0, 0)),
        ],
        out_specs=pl.BlockSpec((BLK1, H), lambda i: (i, 0)),
        out_shape=jax.ShapeDtypeStruct((N, H), jnp.float32),
    )(e_i, w)


def _wap_body(a_ref, b_ref, o_ref):
    o_ref[...] = jnp.dot(a_ref[...], b_ref[...],
                         preferred_element_type=jnp.float32)


def _wap(W_a2, W_p1):
    return pl.pallas_call(
        _wap_body,
        out_shape=jax.ShapeDtypeStruct((H, H), jnp.float32),
    )(W_a2, W_p1)


_GATHER_DNUMS = lax.GatherDimensionNumbers(
    offset_dims=(), collapsed_slice_dims=(0,), start_index_map=(0,))


def _sc_body(g_hbm, src_hbm, dst_hbm, w_hbm, z_hbm, out_hbm,
             srcb, dstb, wb, rowsA, rowsB, acc,
             gsem0, gsem1, ssem0, ssem1):
    cid = lax.axis_index("c")
    sid = lax.axis_index("s")
    wid = sid * NC + cid

    base_big = sid * ZR_BIG
    base_small = 2 * ZR_BIG + (sid - 2) * ZR_SMALL

    @pl.when(sid < 2)
    def _():
        pltpu.sync_copy(z_hbm, acc.at[pl.ds(pl.multiple_of(base_big, 8),
                                            ZR_BIG)])

    @pl.when(sid >= 2)
    def _():
        pltpu.sync_copy(z_hbm.at[pl.ds(0, ZR_SMALL)],
                        acc.at[pl.ds(pl.multiple_of(base_small, 8), ZR_SMALL)])

    plsc.subcore_barrier()

    rows2 = (rowsA, rowsB)
    gsem2 = (gsem0, gsem1)
    ssem2 = (ssem0, ssem1)

    def scale(buf, j):
        def grp(i, c2):
            w16 = wb[j, pl.ds(i * 16, 16)]
            for jj in range(16):
                wbc = lax.gather(
                    w16, jnp.full((16, 1), jj, jnp.int32), _GATHER_DNUMS, (1,),
                    mode=lax.GatherScatterMode.PROMISE_IN_BOUNDS)
                for kk in range(H // 16):
                    sl = buf[i * 16 + jj, pl.ds(kk * 16, 16)]
                    buf[i * 16 + jj, pl.ds(kk * 16, 16)] = sl * wbc
            return c2

        lax.fori_loop(0, C // 16, grp, 0)

    def super_body(s, carry):
        csup = pl.multiple_of((wid * NSUP + s) * SUP, SUP)
        pltpu.sync_copy(src_hbm.at[pl.ds(csup, SUP)], srcb)
        pltpu.sync_copy(dst_hbm.at[pl.ds(csup, SUP)], dstb)
        pltpu.sync_copy(w_hbm.at[pl.ds(csup, SUP)], wb)
        pltpu.async_copy(g_hbm.at[srcb.at[0]], rowsA, gsem0)
        for j in range(SUP):
            cur = j % 2
            nxt = (j + 1) % 2
            if j + 1 < SUP:
                if j >= 1:
                    pltpu.make_async_copy(rows2[nxt], acc.at[dstb.at[j - 1]],
                                          ssem2[nxt]).wait()
                pltpu.async_copy(g_hbm.at[srcb.at[j + 1]], rows2[nxt],
                                 gsem2[nxt])
            pltpu.make_async_copy(g_hbm.at[srcb.at[j]], rows2[cur],
                                  gsem2[cur]).wait()
            scale(rows2[cur], j)
            pltpu.async_copy(rows2[cur], acc.at[dstb.at[j]], ssem2[cur],
                             add=True)
        pltpu.make_async_copy(rows2[0], acc.at[dstb.at[SUP - 2]],
                              ssem2[0]).wait()
        pltpu.make_async_copy(rows2[1], acc.at[dstb.at[SUP - 1]],
                              ssem2[1]).wait()
        return carry

    lax.fori_loop(0, NSUP, super_body, 0)
    plsc.subcore_barrier()

    @pl.when(sid < 2)
    def _():
        b = pl.multiple_of(base_big, 8)
        pltpu.sync_copy(acc.at[pl.ds(b, ZR_BIG)],
                        out_hbm.at[cid, pl.ds(b, ZR_BIG)])

    @pl.when(sid >= 2)
    def _():
        b = pl.multiple_of(base_small, 8)
        pltpu.sync_copy(acc.at[pl.ds(b, ZR_SMALL)],
                        out_hbm.at[cid, pl.ds(b, ZR_SMALL)])


@functools.cache
def _get_sc_segsum():
    mesh = plsc.VectorSubcoreMesh(core_axis_name="c", subcore_axis_name="s")
    return pl.kernel(
        _sc_body,
        mesh=mesh,
        out_type=jax.ShapeDtypeStruct((NC, N, H), jnp.float32),
        scratch_types=[
            pltpu.VMEM((SUP, C), jnp.int32),
            pltpu.VMEM((SUP, C), jnp.int32),
            pltpu.VMEM((SUP, C), jnp.float32),
            pltpu.VMEM((C, H), jnp.float32),
            pltpu.VMEM((C, H), jnp.float32),
            pltpu.VMEM_SHARED((N, H), jnp.float32),
            pltpu.SemaphoreType.DMA,
            pltpu.SemaphoreType.DMA,
            pltpu.SemaphoreType.DMA,
            pltpu.SemaphoreType.DMA,
        ],
    )


BLK2 = 2000


def _tc2_body(a0_ref, a1_ref, q_ref, bg1_ref, wa1_ref, ba1_ref,
              wap_ref, bp1_ref, wp2_ref, bp2_ref, out_ref):
    h = jnp.maximum(a0_ref[...] + a1_ref[...] + bg1_ref[...], 0.0)
    z = jnp.maximum(
        jnp.dot(h, wa1_ref[...], preferred_element_type=jnp.float32)
        + ba1_ref[...], 0.0)
    p = jnp.maximum(
        q_ref[...]
        + jnp.dot(z, wap_ref[...], preferred_element_type=jnp.float32)
        + bp1_ref[...], 0.0)
    out_ref[...] = (jnp.dot(p, wp2_ref[...], preferred_element_type=jnp.float32)
                    + bp2_ref[...])


def _tc2(a0, a1, q, b_g1, W_a1, b_a1, wap, b_p1, W_p2, b_p2):
    row = lambda i: (i, 0)
    full = lambda i: (0, 0)
    return pl.pallas_call(
        _tc2_body,
        grid=(N // BLK2,),
        in_specs=[
            pl.BlockSpec((BLK2, H), row),
            pl.BlockSpec((BLK2, H), row),
            pl.BlockSpec((BLK2, H), row),
            pl.BlockSpec((1, H), full),
            pl.BlockSpec((H, H), full),
            pl.BlockSpec((1, H), full),
            pl.BlockSpec((H, H), full),
            pl.BlockSpec((1, H), full),
            pl.BlockSpec((H, FH), full),
            pl.BlockSpec((1, FH), full),
        ],
        out_specs=pl.BlockSpec((BLK2, FH), row),
        out_shape=jax.ShapeDtypeStruct((N, FH), jnp.float32),
    )(a0, a1, q, b_g1, W_a1, b_a1, wap, b_p1, W_p2, b_p2)


def kernel(e_i, edge_index, edge_weight, W_g1, b_g1, W_a1, b_a1, W_a2,
           W_p1, b_p1, W_p2, b_p2):
    g = _proj(e_i, W_g1)
    q = _proj(e_i, W_p1)
    wap = _wap(W_a2, W_p1)

    zrows = jnp.zeros((ZR_BIG, H), jnp.float32)
    pad = EP - E
    spread = jnp.arange(pad, dtype=jnp.int32)
    src_p = jnp.concatenate([edge_index[0], spread]).reshape(NCHUNK, C)
    dst_p = jnp.concatenate([edge_index[1], spread]).reshape(NCHUNK, C)
    w_p = jnp.pad(edge_weight, (0, pad)).reshape(NCHUNK, C)
    parts = _get_sc_segsum()(g, src_p, dst_p, w_p, zrows)

    return _tc2(parts[0], parts[1], q,
                b_g1.reshape(1, H), W_a1, b_a1.reshape(1, H), wap,
                b_p1.reshape(1, H), W_p2, b_p2.reshape(1, FH))
